# baseline (speedup 1.0000x reference)
import functools

import jax
import jax.numpy as jnp
from jax.experimental import pallas as pl
from jax.experimental.pallas import tpu as pltpu

def _make_chunks(M):
    if M == 9216:
        sizes = (256, 512, 1024, 2048, 2560, 2816)
    else:
        sizes, rem = [], M
        for s in (256, 512, 1024, 2048):
            if rem <= 0:
                break
            sizes.append(min(s, rem))
            rem -= sizes[-1]
        while rem > 0:
            sizes.append(min(2816, rem))
            rem -= sizes[-1]
        sizes = tuple(sizes)
    offs, r0 = [], 0
    for s in sizes:
        offs.append((r0, s))
        r0 += s
    return tuple(offs)


def _dist_kernel(chunks, f_hbm, c_ref, o_hbm, fbuf0, fbuf1, fbuf2, obuf0,
                 obuf1, obuf2, cs_buf, ld_sem, st_sem):
    fbufs = (fbuf0, fbuf1, fbuf2)
    obufs = (obuf0, obuf1, obuf2)

    loads = [
        pltpu.make_async_copy(
            f_hbm.at[pl.ds(r0, sz), :],
            fbufs[i % 3].at[pl.ds(0, sz), :],
            ld_sem.at[i % 3],
        )
        for i, (r0, sz) in enumerate(chunks)
    ]
    loads[0].start()
    if len(loads) > 1:
        loads[1].start()

    c = c_ref[...]
    cs_buf[...] = (-2.0 * c).astype(jnp.bfloat16)
    c2 = jnp.sum(c * c, axis=1)[None, :]

    stores = []
    for i, (r0, sz) in enumerate(chunks):
        if i + 2 < len(chunks):
            loads[i + 2].start()
        loads[i].wait()
        if i >= 3:
            stores[i - 3].wait()
        f = fbufs[i % 3][pl.ds(0, sz), :]
        f2 = jnp.sum(f * f, axis=1, keepdims=True)
        fc = jax.lax.dot_general(
            f.astype(jnp.bfloat16), cs_buf[...],
            (((1,), (1,)), ((), ())),
            preferred_element_type=jnp.float32,
        )
        obufs[i % 3][pl.ds(0, sz), :] = (fc + f2) + c2
        st = pltpu.make_async_copy(
            obufs[i % 3].at[pl.ds(0, sz), :],
            o_hbm.at[pl.ds(r0, sz), :],
            st_sem.at[i % 3],
        )
        st.start()
        stores.append(st)
    for st in stores[-3:]:
        st.wait()


@jax.jit
def _dists(f, c):
    M, D = f.shape
    K = c.shape[0]
    chunks = _make_chunks(M)
    maxrows = max(sz for _, sz in chunks)
    return pl.pallas_call(
        functools.partial(_dist_kernel, chunks),
        in_specs=[
            pl.BlockSpec(memory_space=pl.ANY),
            pl.BlockSpec((K, D), lambda: (0, 0)),
        ],
        out_specs=pl.BlockSpec(memory_space=pl.ANY),
        out_shape=jax.ShapeDtypeStruct((M, K), jnp.float32),
        scratch_shapes=[
            pltpu.VMEM((maxrows, D), jnp.float32),
            pltpu.VMEM((maxrows, D), jnp.float32),
            pltpu.VMEM((maxrows, D), jnp.float32),
            pltpu.VMEM((maxrows, K), jnp.float32),
            pltpu.VMEM((maxrows, K), jnp.float32),
            pltpu.VMEM((maxrows, K), jnp.float32),
            pltpu.VMEM((K, D), jnp.bfloat16),
            pltpu.SemaphoreType.DMA((3,)),
            pltpu.SemaphoreType.DMA((3,)),
        ],
    )(f, c)


def kernel(features, Ck):
    B, S, D = features.shape
    K = Ck.shape[1]
    f = features.reshape(B * S, D)
    c = Ck.reshape(K, D)
    dists = _dists(f, c)
    return dists.reshape(B, S, K)
